# Initial kernel scaffold; baseline (speedup 1.0000x reference)
import jax
import jax.numpy as jnp
from jax.experimental import pallas as pl


def kernel(z, e):
    raise NotImplementedError("write your pallas kernel here")


# speedup vs baseline: 2.0934x; 2.0934x over previous
import jax
import jax.numpy as jnp
from jax.experimental import pallas as pl


def _latent_body(z_ref, zt_ref, e_ref, e2_ref, mask_ref,
                 znew_ref, minloss_ref, wise_ref):
    z = z_ref[...]
    mask = mask_ref[...]
    znew_ref[...] = z * mask

    e = e_ref[...]
    zsq = jnp.sum(z * z, axis=1, keepdims=True)
    esq = jnp.sum(e * e, axis=1, keepdims=True).T
    g = jax.lax.dot_general(z, e, (((1,), (1,)), ((), ())),
                            preferred_element_type=jnp.float32,
                            precision=jax.lax.Precision.HIGHEST)
    d2 = (zsq - 2.0 * g) + esq
    colmin = jnp.min(d2, axis=0, keepdims=True)
    minloss_ref[...] = jnp.sum(colmin, axis=1, keepdims=True) / colmin.shape[1]

    e2 = e2_ref[...]
    nb = 16
    n_total = zt_ref.shape[0]

    def body(i, acc):
        zc = zt_ref[pl.ds(i * nb, nb), :]
        for j in range(nb):
            t = e2 - zc[j:j + 1, :]
            acc = jnp.minimum(acc, t * t)
        return acc

    acc0 = jnp.full(e2.shape, jnp.inf, dtype=jnp.float32)
    acc = jax.lax.fori_loop(0, n_total // nb, body, acc0)
    s = jnp.sum(acc, axis=1, keepdims=True)
    wise_ref[...] = jnp.sum(s, axis=0, keepdims=True) / (e.shape[0] * e.shape[1])


def kernel(z, e):
    n, d = z.shape
    k = e.shape[0]
    k1, k2 = jax.random.split(jax.random.key(42))
    probs = jax.random.uniform(k1, (n,), dtype=z.dtype)
    dropout = jax.random.uniform(k2, z.shape, dtype=z.dtype)
    mask = (dropout < probs[:, None]).astype(z.dtype)

    zt = jnp.concatenate([z, z], axis=1)
    e2 = e.reshape(k // 2, 2 * d)

    znew, minloss, wise = pl.pallas_call(
        _latent_body,
        out_shape=(
            jax.ShapeDtypeStruct((n, d), jnp.float32),
            jax.ShapeDtypeStruct((1, 1), jnp.float32),
            jax.ShapeDtypeStruct((1, 1), jnp.float32),
        ),
    )(z, zt, e, e2, mask)
    return znew, minloss[0, 0], wise[0, 0]
